# baseline (speedup 1.0000x reference)
import functools

import jax
import jax.numpy as jnp
from jax import lax
from jax.experimental import pallas as pl
from jax.experimental.pallas import tpu as pltpu
from jax.experimental.pallas import tpu_sc as plsc

E = 8
D = 1024
H = 1024
N = 8192

BLK = 256
NB = N // BLK + E
NPAD = NB * BLK

NC = 2
NS = 16
NW = NC * NS
GCH = 64


def _routing(ids):
    oh = (ids[:, None] == jnp.arange(E, dtype=jnp.int32)[None, :])
    csum = jnp.cumsum(oh.astype(jnp.int32), axis=0)
    counts = csum[-1]
    rank = jnp.take_along_axis(csum, ids[:, None], axis=1)[:, 0] - 1
    bcounts = (counts + BLK - 1) // BLK
    block_ends = jnp.cumsum(bcounts)
    padded_starts = (block_ends - bcounts) * BLK
    pos = (padded_starts[ids] + rank).astype(jnp.int32)
    blk_ids = jnp.arange(NB, dtype=jnp.int32)
    block_expert = jnp.sum(
        (blk_ids[:, None] >= block_ends[None, :]).astype(jnp.int32), axis=1)
    block_expert = jnp.minimum(block_expert, E - 1).astype(jnp.int32)
    return pos, block_expert


@functools.cache
def _sc_kernels():
    mesh = plsc.VectorSubcoreMesh(
        core_axis_name="c", subcore_axis_name="s",
        num_cores=NC, num_subcores=NS)

    @functools.partial(
        pl.kernel,
        out_type=jax.ShapeDtypeStruct((NPAD, D), jnp.float32),
        mesh=mesh,
        scratch_types=[
            pltpu.VMEM((GCH,), jnp.int32),
            pltpu.VMEM((GCH, D), jnp.float32),
            pltpu.SemaphoreType.DMA,
        ],
    )
    def sc_dispatch(pos_hbm, x_hbm, out_hbm, idx_v, rows_v, sem):
        wid = lax.axis_index("s") * NC + lax.axis_index("c")
        base = wid * (N // NW)

        def step(i, carry):
            off = base + i * GCH
            pltpu.sync_copy(pos_hbm.at[pl.ds(off, GCH)], idx_v)
            pltpu.sync_copy(x_hbm.at[pl.ds(off, GCH)], rows_v)
            pltpu.async_copy(rows_v, out_hbm.at[idx_v], sem).wait()
            return carry

        lax.fori_loop(0, N // NW // GCH, step, 0)

    @functools.partial(
        pl.kernel,
        out_type=(
            jax.ShapeDtypeStruct((N, H), jnp.float32),
            jax.ShapeDtypeStruct((N,), jnp.float32),
        ),
        mesh=mesh,
        scratch_types=[
            pltpu.VMEM((GCH,), jnp.int32),
            pltpu.VMEM((GCH, H), jnp.float32),
            pltpu.VMEM((GCH,), jnp.float32),
            pltpu.SemaphoreType.DMA,
            pltpu.SemaphoreType.DMA,
        ],
    )
    def sc_combine(pos_hbm, act_hbm, val_hbm, out_act_hbm, out_val_hbm,
                   idx_v, rows_v, vals_v, sem_a, sem_v):
        wid = lax.axis_index("s") * NC + lax.axis_index("c")
        base = wid * (N // NW)

        def step(i, carry):
            off = base + i * GCH
            pltpu.sync_copy(pos_hbm.at[pl.ds(off, GCH)], idx_v)
            cp_a = pltpu.async_copy(act_hbm.at[idx_v], rows_v, sem_a)
            cp_v = pltpu.async_copy(val_hbm.at[idx_v], vals_v, sem_v)
            cp_a.wait()
            cp_v.wait()
            pltpu.sync_copy(rows_v, out_act_hbm.at[pl.ds(off, GCH)])
            pltpu.sync_copy(vals_v, out_val_hbm.at[pl.ds(off, GCH)])
            return carry

        lax.fori_loop(0, N // NW // GCH, step, 0)

    return sc_dispatch, sc_combine


def _tc_body(be_ref, vb_ref, x_ref, aW1_ref, ab1_ref, aW2_ref, ab2_ref,
             cW1_ref, cb1_ref, cW2_ref, cb2_ref, vW_ref, act_ref, val_ref):
    x = x_ref[...].astype(jnp.bfloat16)
    ha = jnp.tanh(jnp.dot(x, aW1_ref[0], preferred_element_type=jnp.float32)
                  + ab1_ref[0])
    act = jnp.tanh(jnp.dot(ha.astype(jnp.bfloat16), aW2_ref[0],
                           preferred_element_type=jnp.float32) + ab2_ref[0])
    hc = jnp.tanh(jnp.dot(x, cW1_ref[0], preferred_element_type=jnp.float32)
                  + cb1_ref[0])
    c2 = jnp.tanh(jnp.dot(hc.astype(jnp.bfloat16), cW2_ref[0],
                          preferred_element_type=jnp.float32) + cb2_ref[0])
    e = be_ref[pl.program_id(0)]
    val = jnp.sum(c2 * vW_ref[0], axis=1, keepdims=True) + vb_ref[e]
    act_ref[...] = act
    val_ref[...] = val


def _tc_experts(block_expert, vb_flat, x_pad, aW1, ab1, aW2, ab2,
                cW1, cb1, cW2, cb2, vW):
    grid_spec = pltpu.PrefetchScalarGridSpec(
        num_scalar_prefetch=2,
        grid=(NB,),
        in_specs=[
            pl.BlockSpec((BLK, D), lambda i, be, vb: (i, 0)),
            pl.BlockSpec((1, D, H), lambda i, be, vb: (be[i], 0, 0)),
            pl.BlockSpec((1, 1, H), lambda i, be, vb: (be[i], 0, 0)),
            pl.BlockSpec((1, H, H), lambda i, be, vb: (be[i], 0, 0)),
            pl.BlockSpec((1, 1, H), lambda i, be, vb: (be[i], 0, 0)),
            pl.BlockSpec((1, D, H), lambda i, be, vb: (be[i], 0, 0)),
            pl.BlockSpec((1, 1, H), lambda i, be, vb: (be[i], 0, 0)),
            pl.BlockSpec((1, H, H), lambda i, be, vb: (be[i], 0, 0)),
            pl.BlockSpec((1, 1, H), lambda i, be, vb: (be[i], 0, 0)),
            pl.BlockSpec((1, 1, H), lambda i, be, vb: (be[i], 0, 0)),
        ],
        out_specs=[
            pl.BlockSpec((BLK, H), lambda i, be, vb: (i, 0)),
            pl.BlockSpec((BLK, 1), lambda i, be, vb: (i, 0)),
        ],
    )
    return pl.pallas_call(
        _tc_body,
        grid_spec=grid_spec,
        out_shape=[
            jax.ShapeDtypeStruct((NPAD, H), jnp.float32),
            jax.ShapeDtypeStruct((NPAD, 1), jnp.float32),
        ],
        compiler_params=pltpu.CompilerParams(
            dimension_semantics=("arbitrary",),
        ),
    )(block_expert, vb_flat,
      x_pad,
      aW1.astype(jnp.bfloat16), ab1.reshape(E, 1, H),
      aW2.astype(jnp.bfloat16), ab2.reshape(E, 1, H),
      cW1.astype(jnp.bfloat16), cb1.reshape(E, 1, H),
      cW2.astype(jnp.bfloat16), cb2.reshape(E, 1, H),
      vW.reshape(E, 1, H))


def kernel(controller_ids, inputs, rnn_hxs, masks, aW1, ab1, aW2, ab2,
           cW1, cb1, cW2, cb2, vW, vb):
    ids = controller_ids.astype(jnp.int32)
    pos, block_expert = _routing(ids)

    sc_dispatch, sc_combine = _sc_kernels()
    x_pad = sc_dispatch(pos, inputs)
    act_pad, val_pad = _tc_experts(
        block_expert, vb.reshape(E), x_pad,
        aW1, ab1, aW2, ab2, cW1, cb1, cW2, cb2, vW)
    actor, value = sc_combine(pos, act_pad, val_pad.reshape(NPAD))

    return (value.reshape(N, 1), actor, rnn_hxs)


# speedup vs baseline: 1.2281x; 1.2281x over previous
import functools

import jax
import jax.numpy as jnp
from jax import lax
from jax.experimental import pallas as pl
from jax.experimental.pallas import tpu as pltpu
from jax.experimental.pallas import tpu_sc as plsc

E = 8
D = 1024
H = 1024
N = 8192

BLK = 256
NB = N // BLK + E
NPAD = NB * BLK

NC = 2
NS = 16
NW = NC * NS
GCH = 64


def _routing(ids):
    oh = (ids[:, None] == jnp.arange(E, dtype=jnp.int32)[None, :])
    csum = jnp.cumsum(oh.astype(jnp.int32), axis=0)
    counts = csum[-1]
    bcounts = (counts + BLK - 1) // BLK
    block_ends = jnp.cumsum(bcounts)
    padded_starts = (block_ends - bcounts) * BLK
    pos = jnp.sum(jnp.where(oh, csum - 1 + padded_starts[None, :], 0),
                  axis=1).astype(jnp.int32)
    blk_ids = jnp.arange(NB, dtype=jnp.int32)
    block_expert = jnp.sum(
        (blk_ids[:, None] >= block_ends[None, :]).astype(jnp.int32), axis=1)
    block_expert = jnp.minimum(block_expert, E - 1).astype(jnp.int32)
    return pos, block_expert


@functools.cache
def _sc_kernels():
    mesh = plsc.VectorSubcoreMesh(
        core_axis_name="c", subcore_axis_name="s",
        num_cores=NC, num_subcores=NS)

    @functools.partial(
        pl.kernel,
        out_type=jax.ShapeDtypeStruct((NPAD, D), jnp.float32),
        mesh=mesh,
        scratch_types=[
            pltpu.VMEM((GCH,), jnp.int32),
            pltpu.VMEM((GCH, D), jnp.float32),
            pltpu.SemaphoreType.DMA,
        ],
    )
    def sc_dispatch(pos_hbm, x_hbm, out_hbm, idx_v, rows_v, sem):
        wid = lax.axis_index("s") * NC + lax.axis_index("c")
        base = wid * (N // NW)

        def step(i, carry):
            off = base + i * GCH
            pltpu.sync_copy(pos_hbm.at[pl.ds(off, GCH)], idx_v)
            pltpu.sync_copy(x_hbm.at[pl.ds(off, GCH)], rows_v)
            pltpu.async_copy(rows_v, out_hbm.at[idx_v], sem).wait()
            return carry

        lax.fori_loop(0, N // NW // GCH, step, 0)

    @functools.partial(
        pl.kernel,
        out_type=(
            jax.ShapeDtypeStruct((N, H), jnp.float32),
            jax.ShapeDtypeStruct((N,), jnp.float32),
        ),
        mesh=mesh,
        scratch_types=[
            pltpu.VMEM((GCH,), jnp.int32),
            pltpu.VMEM((GCH, H), jnp.float32),
            pltpu.VMEM((GCH,), jnp.float32),
            pltpu.SemaphoreType.DMA,
            pltpu.SemaphoreType.DMA,
        ],
    )
    def sc_combine(pos_hbm, act_hbm, val_hbm, out_act_hbm, out_val_hbm,
                   idx_v, rows_v, vals_v, sem_a, sem_v):
        wid = lax.axis_index("s") * NC + lax.axis_index("c")
        base = wid * (N // NW)

        def step(i, carry):
            off = base + i * GCH
            pltpu.sync_copy(pos_hbm.at[pl.ds(off, GCH)], idx_v)
            cp_a = pltpu.async_copy(act_hbm.at[idx_v], rows_v, sem_a)
            cp_v = pltpu.async_copy(val_hbm.at[idx_v], vals_v, sem_v)
            cp_a.wait()
            cp_v.wait()
            pltpu.sync_copy(rows_v, out_act_hbm.at[pl.ds(off, GCH)])
            pltpu.sync_copy(vals_v, out_val_hbm.at[pl.ds(off, GCH)])
            return carry

        lax.fori_loop(0, N // NW // GCH, step, 0)

    return sc_dispatch, sc_combine


def _tc_body(be_ref, vb_ref, x_ref, aW1_ref, ab1_ref, aW2_ref, ab2_ref,
             cW1_ref, cb1_ref, cW2_ref, cb2_ref, vW_ref, act_ref, val_ref):
    x = x_ref[...]
    ha = jnp.tanh(jnp.dot(x, aW1_ref[0], preferred_element_type=jnp.float32)
                  + ab1_ref[0])
    act = jnp.tanh(jnp.dot(ha, aW2_ref[0], preferred_element_type=jnp.float32)
                   + ab2_ref[0])
    hc = jnp.tanh(jnp.dot(x, cW1_ref[0], preferred_element_type=jnp.float32)
                  + cb1_ref[0])
    c2 = jnp.tanh(jnp.dot(hc, cW2_ref[0], preferred_element_type=jnp.float32)
                  + cb2_ref[0])
    e = be_ref[pl.program_id(0)]
    val = jnp.sum(c2 * vW_ref[0], axis=1, keepdims=True) + vb_ref[e]
    act_ref[...] = act
    val_ref[...] = val


def _tc_experts(block_expert, vb_flat, x_pad, aW1, ab1, aW2, ab2,
                cW1, cb1, cW2, cb2, vW):
    grid_spec = pltpu.PrefetchScalarGridSpec(
        num_scalar_prefetch=2,
        grid=(NB,),
        in_specs=[
            pl.BlockSpec((BLK, D), lambda i, be, vb: (i, 0)),
            pl.BlockSpec((1, D, H), lambda i, be, vb: (be[i], 0, 0)),
            pl.BlockSpec((1, 1, H), lambda i, be, vb: (be[i], 0, 0)),
            pl.BlockSpec((1, H, H), lambda i, be, vb: (be[i], 0, 0)),
            pl.BlockSpec((1, 1, H), lambda i, be, vb: (be[i], 0, 0)),
            pl.BlockSpec((1, D, H), lambda i, be, vb: (be[i], 0, 0)),
            pl.BlockSpec((1, 1, H), lambda i, be, vb: (be[i], 0, 0)),
            pl.BlockSpec((1, H, H), lambda i, be, vb: (be[i], 0, 0)),
            pl.BlockSpec((1, 1, H), lambda i, be, vb: (be[i], 0, 0)),
            pl.BlockSpec((1, 1, H), lambda i, be, vb: (be[i], 0, 0)),
        ],
        out_specs=[
            pl.BlockSpec((BLK, H), lambda i, be, vb: (i, 0)),
            pl.BlockSpec((BLK, 1), lambda i, be, vb: (i, 0)),
        ],
    )
    return pl.pallas_call(
        _tc_body,
        grid_spec=grid_spec,
        out_shape=[
            jax.ShapeDtypeStruct((NPAD, H), jnp.float32),
            jax.ShapeDtypeStruct((NPAD, 1), jnp.float32),
        ],
        compiler_params=pltpu.CompilerParams(
            dimension_semantics=("arbitrary",),
        ),
    )(block_expert, vb_flat,
      x_pad,
      aW1, ab1.reshape(E, 1, H),
      aW2, ab2.reshape(E, 1, H),
      cW1, cb1.reshape(E, 1, H),
      cW2, cb2.reshape(E, 1, H),
      vW.reshape(E, 1, H))


def kernel(controller_ids, inputs, rnn_hxs, masks, aW1, ab1, aW2, ab2,
           cW1, cb1, cW2, cb2, vW, vb):
    ids = controller_ids.astype(jnp.int32)
    pos, block_expert = _routing(ids)

    sc_dispatch, sc_combine = _sc_kernels()
    x_pad = sc_dispatch(pos, inputs)
    act_pad, val_pad = _tc_experts(
        block_expert, vb.reshape(E), x_pad,
        aW1, ab1, aW2, ab2, cW1, cb1, cW2, cb2, vW)
    actor, value = sc_combine(pos, act_pad, val_pad.reshape(NPAD))

    return (value.reshape(N, 1), actor, rnn_hxs)
